# baseline (speedup 1.0000x reference)
import functools

import jax
import jax.numpy as jnp
from jax import lax
from jax.experimental import pallas as pl
from jax.experimental.pallas import tpu as pltpu
from jax.experimental.pallas import tpu_sc as plsc

BATCH = 16384
D = 64
L = 16
LB = 128
NC = 2
NS = 16
NW = NC * NS
BATCH_TC = 4096
BATCH_SC = BATCH - BATCH_TC
BPW = BATCH_SC // NW
NSLOT = 4


def _sc_body(idx_hbm, uft_hbm, ift_hbm, out_hbm, idx_v, ubuf, vbuf, out_v,
             *sems):
    wid = lax.axis_index("s") * NC + lax.axis_index("c")
    base = wid * BPW

    pltpu.sync_copy(idx_hbm.at[wid], idx_v.at[pl.ds(0, 2 * BPW)])

    def fire(i, slot):
        vec = idx_v[pl.ds(i, L)]
        ru = vec[0]
        vec2 = idx_v[pl.ds(BPW + i, L)]
        rv = vec2[0]
        su = pl.multiple_of((ru >> 7) * LB, LB)
        sv = pl.multiple_of((rv >> 7) * LB, LB)
        pltpu.async_copy(uft_hbm.at[:, pl.ds(su, LB)], ubuf.at[slot],
                         sems[slot])
        pltpu.async_copy(ift_hbm.at[:, pl.ds(sv, LB)], vbuf.at[slot],
                         sems[slot])

    def wait_pair(slot):
        pltpu.make_async_copy(uft_hbm.at[:, pl.ds(0, LB)], ubuf.at[slot],
                              sems[slot]).wait()
        pltpu.make_async_copy(ift_hbm.at[:, pl.ds(0, LB)], vbuf.at[slot],
                              sems[slot]).wait()

    for s in range(NSLOT):
        fire(s, s)

    lane = lax.iota(jnp.int32, L)
    perms = {k: lane ^ k for k in (1, 2, 4, 8)}

    def permute(x, p):
        return x.at[p].get(mode="promise_in_bounds")

    def merge(a, b, k):
        ta = a + permute(a, perms[k])
        tb = b + permute(b, perms[k])
        return jnp.where((lane & k) == 0, ta, tb)

    iot = lax.iota(jnp.int32, L)

    def group_body(g, carry):
        uvec = idx_v[pl.ds(g * L, L)]
        ivec = idx_v[pl.ds(BPW + g * L, L)]
        rows = []
        for j in range(L):
            i = g * L + j
            slot = j % NSLOT
            wait_pair(slot)
            rlu = jnp.full((L,), uvec[j] & 127, jnp.int32)
            rlv = jnp.full((L,), ivec[j] & 127, jnp.int32)
            sl = jnp.full((L,), slot, jnp.int32)
            acc = None
            for k in range(D // L):
                cs = iot + (k * L)
                u = plsc.load_gather(ubuf, [sl, cs, rlu])
                v = plsc.load_gather(vbuf, [sl, cs, rlv])
                acc = u * v if acc is None else acc + u * v
            rows.append(acc)
            fire(jnp.minimum(i + NSLOT, BPW - 1), slot)
        for k in (1, 2, 4, 8):
            rows = [merge(rows[2 * m], rows[2 * m + 1], k)
                    for m in range(len(rows) // 2)]
        out_v[pl.ds(g * L, L)] = rows[0]
        return carry

    lax.fori_loop(0, BPW // L, group_body, 0)

    for s in range(NSLOT):
        wait_pair(s)

    pltpu.sync_copy(out_v, out_hbm.at[pl.ds(base, BPW)])


def _tc_body(idx_ref, ublock, vblock, out_ref):
    g1 = pl.program_id(1)
    step = pl.program_id(0) * LB + g1
    rl_u = idx_ref[0, step] & 127
    rl_v = idx_ref[1, step] & 127
    lane = lax.broadcasted_iota(jnp.int32, (D, LB), 1)
    u = jnp.sum(jnp.where(lane == rl_u, ublock[:], 0.0), axis=1,
                keepdims=True)
    v = jnp.sum(jnp.where(lane == rl_v, vblock[:], 0.0), axis=1,
                keepdims=True)
    s = jnp.sum(u * v)
    lane1 = lax.broadcasted_iota(jnp.int32, (1, 1, LB), 2)
    prev = jnp.where(g1 == 0, jnp.zeros_like(out_ref[:]), out_ref[:])
    out_ref[:] = jnp.where(lane1 == g1, s, prev)


def _tc_gather(idx_tc, uft, ift):
    grid_spec = pltpu.PrefetchScalarGridSpec(
        num_scalar_prefetch=1,
        grid=(BATCH_TC // LB, LB),
        in_specs=[
            pl.BlockSpec((D, LB),
                         lambda g0, g1, idx: (0, idx[0, g0 * LB + g1] >> 7)),
            pl.BlockSpec((D, LB),
                         lambda g0, g1, idx: (0, idx[1, g0 * LB + g1] >> 7)),
        ],
        out_specs=pl.BlockSpec((1, 1, LB), lambda g0, g1, idx: (g0, 0, 0)),
    )
    out = pl.pallas_call(
        _tc_body,
        grid_spec=grid_spec,
        out_shape=jax.ShapeDtypeStruct((BATCH_TC // LB, 1, LB), jnp.float32),
    )(idx_tc, uft, ift)
    return out.reshape(BATCH_TC)


@jax.jit
def kernel(user, item, user_factors, item_factors):
    uft = user_factors.T
    ift = item_factors.T
    idx = jnp.concatenate(
        [user[:BATCH_SC].reshape(NW, BPW),
         item[:BATCH_SC].reshape(NW, BPW)], axis=1)
    idx_tc = jnp.stack([user[BATCH_SC:], item[BATCH_SC:]])

    mesh = plsc.VectorSubcoreMesh(core_axis_name="c", subcore_axis_name="s")
    run = functools.partial(
        pl.kernel,
        mesh=mesh,
        compiler_params=pltpu.CompilerParams(needs_layout_passes=False),
        out_type=jax.ShapeDtypeStruct((BATCH_SC,), jnp.float32),
        scratch_types=[
            pltpu.VMEM((2 * BPW + L,), jnp.int32),
            pltpu.VMEM((NSLOT, D, LB), jnp.float32),
            pltpu.VMEM((NSLOT, D, LB), jnp.float32),
            pltpu.VMEM((BPW,), jnp.float32),
        ] + [pltpu.SemaphoreType.DMA] * NSLOT,
    )(_sc_body)
    out_sc = run(idx, uft, ift)
    out_tc = _tc_gather(idx_tc, uft, ift)
    return jnp.concatenate([out_sc, out_tc])


# speedup vs baseline: 5.5640x; 5.5640x over previous
import functools

import jax
import jax.numpy as jnp
from jax import lax
from jax.experimental import pallas as pl
from jax.experimental.pallas import tpu as pltpu
from jax.experimental.pallas import tpu_sc as plsc

BATCH = 16384
D = 64
L = 16
LB = 128
NC = 2
NS = 16
NW = NC * NS
BPW = BATCH // NW
NSLOT = 4


def _sc_body(idx_hbm, uft_hbm, ift_hbm, out_hbm, idx_v, ubuf, vbuf, out_v,
             *sems):
    wid = lax.axis_index("s") * NC + lax.axis_index("c")
    base = wid * BPW

    pltpu.sync_copy(idx_hbm.at[wid], idx_v.at[pl.ds(0, 2 * BPW)])

    def fire(i, slot):
        vec = idx_v[pl.ds(i, L)]
        ru = vec[0]
        vec2 = idx_v[pl.ds(BPW + i, L)]
        rv = vec2[0]
        su = pl.multiple_of((ru >> 7) * LB, LB)
        sv = pl.multiple_of((rv >> 7) * LB, LB)
        pltpu.async_copy(uft_hbm.at[:, pl.ds(su, LB)], ubuf.at[slot],
                         sems[slot])
        pltpu.async_copy(ift_hbm.at[:, pl.ds(sv, LB)], vbuf.at[slot],
                         sems[slot])

    def wait_pair(slot):
        pltpu.make_async_copy(uft_hbm.at[:, pl.ds(0, LB)], ubuf.at[slot],
                              sems[slot]).wait()
        pltpu.make_async_copy(ift_hbm.at[:, pl.ds(0, LB)], vbuf.at[slot],
                              sems[slot]).wait()

    for s in range(NSLOT):
        fire(s, s)

    lane = lax.iota(jnp.int32, L)
    perms = {k: lane ^ k for k in (1, 2, 4, 8)}

    def permute(x, p):
        return x.at[p].get(mode="promise_in_bounds")

    def merge(a, b, k):
        ta = a + permute(a, perms[k])
        tb = b + permute(b, perms[k])
        return jnp.where((lane & k) == 0, ta, tb)

    iot = lax.iota(jnp.int32, L)

    def group_body(g, carry):
        uvec = idx_v[pl.ds(g * L, L)]
        ivec = idx_v[pl.ds(BPW + g * L, L)]
        rows = []
        for j in range(L):
            i = g * L + j
            slot = j % NSLOT
            wait_pair(slot)
            rlu = jnp.full((L,), uvec[j] & 127, jnp.int32)
            rlv = jnp.full((L,), ivec[j] & 127, jnp.int32)
            sl = jnp.full((L,), slot, jnp.int32)
            acc = None
            for k in range(D // L):
                cs = iot + (k * L)
                u = plsc.load_gather(ubuf, [sl, cs, rlu])
                v = plsc.load_gather(vbuf, [sl, cs, rlv])
                acc = u * v if acc is None else acc + u * v
            rows.append(acc)
            fire(jnp.minimum(i + NSLOT, BPW - 1), slot)
        for k in (1, 2, 4, 8):
            rows = [merge(rows[2 * m], rows[2 * m + 1], k)
                    for m in range(len(rows) // 2)]
        out_v[pl.ds(g * L, L)] = rows[0]
        return carry

    lax.fori_loop(0, BPW // L, group_body, 0)

    for s in range(NSLOT):
        wait_pair(s)

    pltpu.sync_copy(out_v, out_hbm.at[pl.ds(base, BPW)])


@jax.jit
def kernel(user, item, user_factors, item_factors):
    uft = user_factors.T
    ift = item_factors.T
    idx = jnp.concatenate(
        [user.reshape(NW, BPW), item.reshape(NW, BPW)], axis=1)

    mesh = plsc.VectorSubcoreMesh(core_axis_name="c", subcore_axis_name="s")
    run = functools.partial(
        pl.kernel,
        mesh=mesh,
        compiler_params=pltpu.CompilerParams(needs_layout_passes=False),
        out_type=jax.ShapeDtypeStruct((BATCH,), jnp.float32),
        scratch_types=[
            pltpu.VMEM((2 * BPW + L,), jnp.int32),
            pltpu.VMEM((NSLOT, D, LB), jnp.float32),
            pltpu.VMEM((NSLOT, D, LB), jnp.float32),
            pltpu.VMEM((BPW,), jnp.float32),
        ] + [pltpu.SemaphoreType.DMA] * NSLOT,
    )(_sc_body)
    return run(idx, uft, ift)
